# baseline (speedup 1.0000x reference)
import functools

import jax
import jax.numpy as jnp
from jax import lax
from jax.experimental import pallas as pl
from jax.experimental.pallas import tpu as pltpu
from jax.experimental.pallas import tpu_sc as plsc

_NC = 2
_NS = 16
_LW = 128
_RING = 10


def _linearize_body(tt_ref, out_ref):
    out_ref[:, 0:16] = jnp.transpose(tt_ref[...], (1, 0))


def _pack_table_tc(emb_table):
    V, EMB = emb_table.shape
    TVL = 2048
    return pl.pallas_call(
        _linearize_body,
        grid=(pl.cdiv(V, TVL),),
        in_specs=[pl.BlockSpec((EMB, TVL), lambda i: (0, i))],
        out_specs=pl.BlockSpec((TVL, _LW), lambda i: (i, 0)),
        out_shape=jax.ShapeDtypeStruct((V, _LW), jnp.float32),
        compiler_params=pltpu.CompilerParams(
            dimension_semantics=("parallel",),
        ),
    )(emb_table.T)


def _embed_pool_sc(context_idxs, table_packed):
    B, CTX = context_idxs.shape
    NW = _NC * _NS
    b_per_w = B // NW
    EMB = 16

    idx_t = context_idxs.astype(jnp.int32).T
    mesh = plsc.VectorSubcoreMesh(core_axis_name="c", subcore_axis_name="s")

    scratch = [pltpu.VMEM((CTX, b_per_w), jnp.int32),
               pltpu.VMEM((b_per_w, EMB), jnp.float32)]
    scratch += [pltpu.VMEM((b_per_w, _LW), jnp.float32)] * _RING
    scratch += [pltpu.SemaphoreType.DMA] * _RING

    @functools.partial(
        pl.kernel,
        mesh=mesh,
        out_type=jax.ShapeDtypeStruct((B, EMB), jnp.float32),
        scratch_types=scratch,
        compiler_params=pltpu.CompilerParams(use_tc_tiling_on_sc=False),
    )
    def gather_pool(table_hbm, idx_hbm, out_hbm, idx_v, emb_v, *rest):
        bufs, sems = rest[:_RING], rest[_RING:]
        wid = lax.axis_index("s") * _NC + lax.axis_index("c")
        base = wid * b_per_w
        pltpu.sync_copy(idx_hbm.at[:, pl.ds(base, b_per_w)], idx_v)

        for j in range(b_per_w):
            emb_v[j, :] = jnp.zeros((EMB,), jnp.float32)

        for p in range(_RING):
            pltpu.make_async_copy(
                table_hbm.at[idx_v.at[p]], bufs[p], sems[p]).start()

        def accumulate(buf):
            for j in range(b_per_w):
                emb_v[j, :] = emb_v[j, :] + buf[j, 0:EMB]

        @pl.loop(0, CTX - _RING, step=_RING)
        def _(co):
            for p in range(_RING):
                pltpu.make_async_copy(
                    table_hbm.at[idx_v.at[co + p]], bufs[p], sems[p]).wait()
                accumulate(bufs[p])
                pltpu.make_async_copy(
                    table_hbm.at[idx_v.at[co + p + _RING]],
                    bufs[p], sems[p]).start()

        for p in range(_RING):
            pltpu.make_async_copy(
                table_hbm.at[idx_v.at[CTX - _RING + p]],
                bufs[p], sems[p]).wait()
            accumulate(bufs[p])

        pltpu.sync_copy(emb_v, out_hbm.at[pl.ds(base, b_per_w)])

    return gather_pool(table_packed, idx_t)


def _mm_body(wt_ref, emb_ref, out_ref):
    out_ref[...] = lax.dot_general(
        wt_ref[...], emb_ref[...],
        dimension_numbers=(((0,), (1,)), ((), ())),
        preferred_element_type=jnp.float32,
    )


def _project_tc(embedded, W, b):
    B, EMB = embedded.shape
    V = W.shape[0]
    K = EMB + 1
    wt_aug = jnp.concatenate([W.T, b[None, :]], axis=0)
    emb_aug = jnp.concatenate(
        [embedded, jnp.ones((B, 1), jnp.float32)], axis=1)
    TV = 2048
    grid = pl.cdiv(V, TV)
    out_t = pl.pallas_call(
        _mm_body,
        grid=(grid,),
        in_specs=[
            pl.BlockSpec((K, TV), lambda i: (0, i)),
            pl.BlockSpec((B, K), lambda i: (0, 0)),
        ],
        out_specs=pl.BlockSpec((TV, B), lambda i: (i, 0)),
        out_shape=jax.ShapeDtypeStruct((V, B), jnp.float32),
        compiler_params=pltpu.CompilerParams(
            dimension_semantics=("parallel",),
        ),
    )(wt_aug, emb_aug)
    return out_t.T


def kernel(context_idxs, emb_table, W, b):
    table_packed = _pack_table_tc(emb_table)
    embedded = _embed_pool_sc(context_idxs, table_packed)
    return _project_tc(embedded, W, b)
